# baseline (speedup 1.0000x reference)
import math

import numpy as _onp

import jax
import jax.numpy as jnp
from jax import lax
from jax.experimental import pallas as pl
from jax.experimental.pallas import tpu as pltpu

_BIG = 1e9
_THRESH = 0.5 * _BIG
_INF = 3.0e9

_RS = 256
_CS = 512
_TN = 2048
_TM = 4096


def _ceil_to(x, m):
    return ((x + m - 1) // m) * m


def _min_fold(vs):
    if len(vs) == 1:
        return vs[0]
    h = len(vs) // 2
    return jnp.minimum(_min_fold(vs[:h]), _min_fold(vs[h:]))


def _hilo(x):
    xi = lax.bitcast_convert_type(x, jnp.uint32)
    hi = lax.bitcast_convert_type(
        (xi + jnp.uint32(0x8000)) & jnp.uint32(0xFFFF0000), jnp.float32)
    lo = x - hi
    return hi.astype(jnp.bfloat16), lo.astype(jnp.bfloat16)


def _pack_features(xyz, val, n_pad, side):
    P, _, N = xyz.shape
    pad = n_pad - N
    if pad:
        xyz = jnp.pad(xyz, ((0, 0), (0, 0), (0, pad)))
        val = jnp.pad(val, ((0, 0), (0, pad)))
    w = xyz if side == "p" else -2.0 * xyz
    sq = jnp.sum(xyz * xyz, axis=1, keepdims=True)
    w_hi, w_lo = _hilo(w)
    s_hi, s_lo = _hilo(sq)
    pen = (_BIG * (1.0 - val))[:, None, :].astype(jnp.bfloat16)
    one = jnp.ones_like(pen)
    zero = jnp.zeros_like(pen)
    if side == "p":
        feats = [w_hi, w_hi, w_lo, s_hi, s_lo, one, one, pen, one, zero]
    else:
        feats = [w_hi, w_lo, w_hi, one, one, s_hi, s_lo, one, pen, zero]
    return jnp.concatenate(feats, axis=1)


def _tile_kernel(cnt_ref, a_ref, b_ref, d1_ref, d2_ref, rmin_ref, qmin_ref):
    p = pl.program_id(0)
    i = pl.program_id(1)
    j = pl.program_id(2)
    last_i = i == pl.num_programs(1) - 1
    last_j = j == pl.num_programs(2) - 1
    tn = a_ref.shape[2]
    tm = b_ref.shape[2]
    nr = tn // _RS
    nc = tm // _CS

    @pl.when((i == 0) & (j == 0))
    def _():
        d1_ref[...] = jnp.zeros_like(d1_ref)

    @pl.when(j == 0)
    def _():
        rmin_ref[...] = jnp.full(rmin_ref.shape, _INF, jnp.bfloat16)

    @pl.when(i == 0)
    def _():
        qmin_ref[:, pl.ds(pl.multiple_of(j * tm, _CS), tm)] = jnp.full(
            (16, tm), _INF, jnp.bfloat16)

    @pl.when((i * tn < cnt_ref[p, 0]) & (j * tm < cnt_ref[p, 1]))
    def _():
        qacc = [qmin_ref[:, pl.ds(pl.multiple_of(j * tm + c * _CS, _CS), _CS)]
                for c in range(nc)]

        for r in range(nr):
            a = a_ref[0, :, r * _RS:(r + 1) * _RS]
            rmin = rmin_ref[r * _RS:(r + 1) * _RS, :]
            for c in range(nc):
                b = b_ref[0, :, c * _CS:(c + 1) * _CS]
                d = lax.dot_general(
                    a, b,
                    dimension_numbers=(((0,), (0,)), ((), ())),
                    preferred_element_type=jnp.float32)
                d = d.astype(jnp.bfloat16)
                rmin = jnp.minimum(rmin, _min_fold(
                    [d[:, k * 128:(k + 1) * 128] for k in range(_CS // 128)]))
                qacc[c] = jnp.minimum(qacc[c], _min_fold(
                    [d[k * 16:(k + 1) * 16, :] for k in range(_RS // 16)]))
            rmin_ref[r * _RS:(r + 1) * _RS, :] = rmin

        for c in range(nc):
            qmin_ref[:, pl.ds(pl.multiple_of(j * tm + c * _CS, _CS), _CS)] = (
                qacc[c])

    @pl.when(last_j)
    def _():
        rm = jnp.min(rmin_ref[...].astype(jnp.float32),
                     axis=1, keepdims=True)
        d1_ref[...] = d1_ref[...] + jnp.reshape(
            jnp.sum(jnp.where(rm < _THRESH, rm, 0.0)), (1, 1, 1))

    @pl.when(last_i & last_j)
    def _():
        qcol = jnp.min(qmin_ref[...].astype(jnp.float32),
                       axis=0, keepdims=True)
        d2_ref[...] = jnp.reshape(
            jnp.sum(jnp.where(qcol < _THRESH, qcol, 0.0)), (1, 1, 1))


def _chamfer_sums_local(p_xyz, p_val, q_xyz, q_val):
    num_pairs, _, n_p = p_xyz.shape
    n_q = q_xyz.shape[2]

    tn = min(_TN, _ceil_to(n_p, _RS))
    tm = min(_TM, _ceil_to(n_q, _CS))
    np_pad = _ceil_to(n_p, tn)
    nq_pad = _ceil_to(n_q, tm)

    lhs = _pack_features(p_xyz, p_val, np_pad, "p")
    rhs = _pack_features(q_xyz, q_val, nq_pad, "q")
    counts = jnp.stack(
        [jnp.sum(p_val, axis=1), jnp.sum(q_val, axis=1)],
        axis=1).astype(jnp.int32)

    grid = (num_pairs, np_pad // tn, nq_pad // tm)
    d1, d2 = pl.pallas_call(
        _tile_kernel,
        out_shape=(jax.ShapeDtypeStruct((num_pairs, 1, 1), jnp.float32),
                   jax.ShapeDtypeStruct((num_pairs, 1, 1), jnp.float32)),
        grid_spec=pltpu.PrefetchScalarGridSpec(
            num_scalar_prefetch=1,
            grid=grid,
            in_specs=[
                pl.BlockSpec((1, 16, tn), lambda p, i, j, c: (p, 0, i)),
                pl.BlockSpec((1, 16, tm), lambda p, i, j, c: (p, 0, j)),
            ],
            out_specs=[
                pl.BlockSpec((1, 1, 1), lambda p, i, j, c: (p, 0, 0)),
                pl.BlockSpec((1, 1, 1), lambda p, i, j, c: (p, 0, 0)),
            ],
            scratch_shapes=[
                pltpu.VMEM((tn, 128), jnp.bfloat16),
                pltpu.VMEM((16, nq_pad), jnp.bfloat16),
            ]),
        compiler_params=pltpu.CompilerParams(
            dimension_semantics=("parallel", "arbitrary", "arbitrary"),
            vmem_limit_bytes=64 * 1024 * 1024),
    )(counts, lhs, rhs)
    return d1[:, 0, 0], d2[:, 0, 0]


def _chamfer_sums(p_xyz, p_val, q_xyz, q_val):
    devs = jax.devices()
    num_pairs = p_xyz.shape[0]
    n_dev = 2 if (len(devs) >= 2 and num_pairs % 2 == 0) else 1
    if n_dev == 1:
        return _chamfer_sums_local(p_xyz, p_val, q_xyz, q_val)
    try:
        shard_map = jax.shard_map
    except AttributeError:
        from jax.experimental.shard_map import shard_map
    mesh = jax.sharding.Mesh(_onp.array(devs[:n_dev]), ("d",))
    spec = jax.sharding.PartitionSpec("d")
    fn = shard_map(_chamfer_sums_local, mesh=mesh,
                   in_specs=(spec, spec, spec, spec),
                   out_specs=(spec, spec), check_vma=False)
    return fn(p_xyz, p_val, q_xyz, q_val)


def _ray_dirs(h, w, fov_up_deg=3.0, fov_down_deg=-25.0):
    fov_up = math.radians(fov_up_deg)
    fov_down = math.radians(fov_down_deg)
    az = -math.pi + (jnp.arange(w, dtype=jnp.float32) + 0.5) * (
        2.0 * math.pi / w)
    el = fov_up - (jnp.arange(h, dtype=jnp.float32) + 0.5) * (
        (fov_up - fov_down) / h)
    az = az[None, :]
    el = el[:, None]
    x = jnp.cos(el) * jnp.cos(az)
    y = jnp.cos(el) * jnp.sin(az)
    z = jnp.sin(el) * jnp.ones_like(az)
    return jnp.stack([x, y, z], axis=0)


def kernel(rv, mask_logits, target):
    rv = rv.astype(jnp.float32)
    logits = mask_logits.astype(jnp.float32)
    target = target.astype(jnp.float32)
    b, t, h, w = rv.shape
    dirs = _ray_dirs(h, w)

    masked_rv = jnp.where(logits > 0.0, rv, -1.0)

    p_xyz = masked_rv[:, :, None] * dirs[None, None]
    p_val = (masked_rv > 0.0).astype(jnp.float32)
    q_xyz = target[:, 1:4]
    q_val = (target[:, 0] > 0.0).astype(jnp.float32)

    p_xyz = p_xyz.transpose(1, 0, 2, 3, 4).reshape(t * b, 3, h * w)
    p_val = p_val.transpose(1, 0, 2, 3).reshape(t * b, h * w)
    q_xyz = q_xyz.transpose(2, 0, 1, 3, 4).reshape(t * b, 3, h * w)
    q_val = q_val.transpose(1, 0, 2, 3).reshape(t * b, h * w)

    n_p = jnp.sum(p_val, axis=1)
    n_q = jnp.sum(q_val, axis=1)

    def _compact(xyz, val):
        _, x, y, z, val = lax.sort(
            (1.0 - val, xyz[:, 0], xyz[:, 1], xyz[:, 2], val),
            dimension=1, is_stable=False, num_keys=1)
        return jnp.stack([x, y, z], axis=1), val

    p_xyz, p_val = _compact(p_xyz, p_val)
    q_xyz, q_val = _compact(q_xyz, q_val)
    d1_sum, d2_sum = _chamfer_sums(p_xyz, p_val, q_xyz, q_val)
    combined = d1_sum / jnp.maximum(n_p, 1.0) + d2_sum / jnp.maximum(n_q, 1.0)
    chamfer_distances_tensor = combined.reshape(t, b)
    chamfer_distances = {s: jnp.mean(chamfer_distances_tensor[s])
                         for s in range(t)}
    return chamfer_distances, chamfer_distances_tensor


# speedup vs baseline: 1.0097x; 1.0097x over previous
import math

import numpy as _onp

import jax
import jax.numpy as jnp
from jax import lax
from jax.experimental import pallas as pl
from jax.experimental.pallas import tpu as pltpu

_BIG = 1e9
_THRESH = 0.5 * _BIG
_INF = 3.0e9

_RS = 256
_CS = 512
_TN = 4096
_TM = 4096


def _ceil_to(x, m):
    return ((x + m - 1) // m) * m


def _min_fold(vs):
    if len(vs) == 1:
        return vs[0]
    h = len(vs) // 2
    return jnp.minimum(_min_fold(vs[:h]), _min_fold(vs[h:]))


def _hilo(x):
    xi = lax.bitcast_convert_type(x, jnp.uint32)
    hi = lax.bitcast_convert_type(
        (xi + jnp.uint32(0x8000)) & jnp.uint32(0xFFFF0000), jnp.float32)
    lo = x - hi
    return hi.astype(jnp.bfloat16), lo.astype(jnp.bfloat16)


def _pack_features(xyz, val, n_pad, side):
    P, _, N = xyz.shape
    pad = n_pad - N
    if pad:
        xyz = jnp.pad(xyz, ((0, 0), (0, 0), (0, pad)))
        val = jnp.pad(val, ((0, 0), (0, pad)))
    w = xyz if side == "p" else -2.0 * xyz
    sq = jnp.sum(xyz * xyz, axis=1, keepdims=True)
    w_hi, w_lo = _hilo(w)
    s_hi, s_lo = _hilo(sq)
    pen = (_BIG * (1.0 - val))[:, None, :].astype(jnp.bfloat16)
    one = jnp.ones_like(pen)
    zero = jnp.zeros_like(pen)
    if side == "p":
        feats = [w_hi, w_hi, w_lo, s_hi, s_lo, one, one, pen, one, zero]
    else:
        feats = [w_hi, w_lo, w_hi, one, one, s_hi, s_lo, one, pen, zero]
    return jnp.concatenate(feats, axis=1)


def _tile_kernel(cnt_ref, a_ref, b_ref, d1_ref, d2_ref, rmin_ref, qmin_ref):
    p = pl.program_id(0)
    i = pl.program_id(1)
    j = pl.program_id(2)
    last_i = i == pl.num_programs(1) - 1
    last_j = j == pl.num_programs(2) - 1
    tn = a_ref.shape[2]
    tm = b_ref.shape[2]
    nr = tn // _RS
    nc = tm // _CS

    @pl.when((i == 0) & (j == 0))
    def _():
        d1_ref[...] = jnp.zeros_like(d1_ref)

    @pl.when(j == 0)
    def _():
        rmin_ref[...] = jnp.full(rmin_ref.shape, _INF, jnp.bfloat16)

    @pl.when(i == 0)
    def _():
        qmin_ref[:, pl.ds(pl.multiple_of(j * tm, _CS), tm)] = jnp.full(
            (16, tm), _INF, jnp.bfloat16)

    @pl.when((i * tn < cnt_ref[p, 0]) & (j * tm < cnt_ref[p, 1]))
    def _():
        qacc = [qmin_ref[:, pl.ds(pl.multiple_of(j * tm + c * _CS, _CS), _CS)]
                for c in range(nc)]

        for r in range(nr):
            a = a_ref[0, :, r * _RS:(r + 1) * _RS]
            rmin = rmin_ref[r * _RS:(r + 1) * _RS, :]
            for c in range(nc):
                b = b_ref[0, :, c * _CS:(c + 1) * _CS]
                d = lax.dot_general(
                    a, b,
                    dimension_numbers=(((0,), (0,)), ((), ())),
                    preferred_element_type=jnp.float32)
                d = d.astype(jnp.bfloat16)
                rmin = jnp.minimum(rmin, _min_fold(
                    [d[:, k * 128:(k + 1) * 128] for k in range(_CS // 128)]))
                qacc[c] = jnp.minimum(qacc[c], _min_fold(
                    [d[k * 16:(k + 1) * 16, :] for k in range(_RS // 16)]))
            rmin_ref[r * _RS:(r + 1) * _RS, :] = rmin

        for c in range(nc):
            qmin_ref[:, pl.ds(pl.multiple_of(j * tm + c * _CS, _CS), _CS)] = (
                qacc[c])

    @pl.when(last_j)
    def _():
        rm = jnp.min(rmin_ref[...].astype(jnp.float32),
                     axis=1, keepdims=True)
        d1_ref[...] = d1_ref[...] + jnp.reshape(
            jnp.sum(jnp.where(rm < _THRESH, rm, 0.0)), (1, 1, 1))

    @pl.when(last_i & last_j)
    def _():
        qcol = jnp.min(qmin_ref[...].astype(jnp.float32),
                       axis=0, keepdims=True)
        d2_ref[...] = jnp.reshape(
            jnp.sum(jnp.where(qcol < _THRESH, qcol, 0.0)), (1, 1, 1))


def _chamfer_sums_local(p_xyz, p_val, q_xyz, q_val):
    num_pairs, _, n_p = p_xyz.shape
    n_q = q_xyz.shape[2]

    tn = min(_TN, _ceil_to(n_p, _RS))
    tm = min(_TM, _ceil_to(n_q, _CS))
    np_pad = _ceil_to(n_p, tn)
    nq_pad = _ceil_to(n_q, tm)

    lhs = _pack_features(p_xyz, p_val, np_pad, "p")
    rhs = _pack_features(q_xyz, q_val, nq_pad, "q")
    counts = jnp.stack(
        [jnp.sum(p_val, axis=1), jnp.sum(q_val, axis=1)],
        axis=1).astype(jnp.int32)

    grid = (num_pairs, np_pad // tn, nq_pad // tm)
    d1, d2 = pl.pallas_call(
        _tile_kernel,
        out_shape=(jax.ShapeDtypeStruct((num_pairs, 1, 1), jnp.float32),
                   jax.ShapeDtypeStruct((num_pairs, 1, 1), jnp.float32)),
        grid_spec=pltpu.PrefetchScalarGridSpec(
            num_scalar_prefetch=1,
            grid=grid,
            in_specs=[
                pl.BlockSpec((1, 16, tn), lambda p, i, j, c: (p, 0, i)),
                pl.BlockSpec((1, 16, tm), lambda p, i, j, c: (p, 0, j)),
            ],
            out_specs=[
                pl.BlockSpec((1, 1, 1), lambda p, i, j, c: (p, 0, 0)),
                pl.BlockSpec((1, 1, 1), lambda p, i, j, c: (p, 0, 0)),
            ],
            scratch_shapes=[
                pltpu.VMEM((tn, 128), jnp.bfloat16),
                pltpu.VMEM((16, nq_pad), jnp.bfloat16),
            ]),
        compiler_params=pltpu.CompilerParams(
            dimension_semantics=("parallel", "arbitrary", "arbitrary"),
            vmem_limit_bytes=64 * 1024 * 1024),
    )(counts, lhs, rhs)
    return d1[:, 0, 0], d2[:, 0, 0]


def _chamfer_sums(p_xyz, p_val, q_xyz, q_val):
    devs = jax.devices()
    num_pairs = p_xyz.shape[0]
    n_dev = 2 if (len(devs) >= 2 and num_pairs % 2 == 0) else 1
    if n_dev == 1:
        return _chamfer_sums_local(p_xyz, p_val, q_xyz, q_val)
    try:
        shard_map = jax.shard_map
    except AttributeError:
        from jax.experimental.shard_map import shard_map
    mesh = jax.sharding.Mesh(_onp.array(devs[:n_dev]), ("d",))
    spec = jax.sharding.PartitionSpec("d")
    fn = shard_map(_chamfer_sums_local, mesh=mesh,
                   in_specs=(spec, spec, spec, spec),
                   out_specs=(spec, spec), check_vma=False)
    return fn(p_xyz, p_val, q_xyz, q_val)


def _ray_dirs(h, w, fov_up_deg=3.0, fov_down_deg=-25.0):
    fov_up = math.radians(fov_up_deg)
    fov_down = math.radians(fov_down_deg)
    az = -math.pi + (jnp.arange(w, dtype=jnp.float32) + 0.5) * (
        2.0 * math.pi / w)
    el = fov_up - (jnp.arange(h, dtype=jnp.float32) + 0.5) * (
        (fov_up - fov_down) / h)
    az = az[None, :]
    el = el[:, None]
    x = jnp.cos(el) * jnp.cos(az)
    y = jnp.cos(el) * jnp.sin(az)
    z = jnp.sin(el) * jnp.ones_like(az)
    return jnp.stack([x, y, z], axis=0)


def kernel(rv, mask_logits, target):
    rv = rv.astype(jnp.float32)
    logits = mask_logits.astype(jnp.float32)
    target = target.astype(jnp.float32)
    b, t, h, w = rv.shape
    dirs = _ray_dirs(h, w)

    masked_rv = jnp.where(logits > 0.0, rv, -1.0)

    p_xyz = masked_rv[:, :, None] * dirs[None, None]
    p_val = (masked_rv > 0.0).astype(jnp.float32)
    q_xyz = target[:, 1:4]
    q_val = (target[:, 0] > 0.0).astype(jnp.float32)

    p_xyz = p_xyz.transpose(1, 0, 2, 3, 4).reshape(t * b, 3, h * w)
    p_val = p_val.transpose(1, 0, 2, 3).reshape(t * b, h * w)
    q_xyz = q_xyz.transpose(2, 0, 1, 3, 4).reshape(t * b, 3, h * w)
    q_val = q_val.transpose(1, 0, 2, 3).reshape(t * b, h * w)

    n_p = jnp.sum(p_val, axis=1)
    n_q = jnp.sum(q_val, axis=1)

    def _compact(xyz, val):
        _, x, y, z, val = lax.sort(
            (1.0 - val, xyz[:, 0], xyz[:, 1], xyz[:, 2], val),
            dimension=1, is_stable=False, num_keys=1)
        return jnp.stack([x, y, z], axis=1), val

    p_xyz, p_val = _compact(p_xyz, p_val)
    q_xyz, q_val = _compact(q_xyz, q_val)
    d1_sum, d2_sum = _chamfer_sums(p_xyz, p_val, q_xyz, q_val)
    combined = d1_sum / jnp.maximum(n_p, 1.0) + d2_sum / jnp.maximum(n_q, 1.0)
    chamfer_distances_tensor = combined.reshape(t, b)
    chamfer_distances = {s: jnp.mean(chamfer_distances_tensor[s])
                         for s in range(t)}
    return chamfer_distances, chamfer_distances_tensor


# speedup vs baseline: 1.0443x; 1.0343x over previous
import math

import numpy as _onp

import jax
import jax.numpy as jnp
from jax import lax
from jax.experimental import pallas as pl
from jax.experimental.pallas import tpu as pltpu

_BIG = 1e9
_THRESH = 0.5 * _BIG
_INF = 3.0e9

_RS = 256
_CS = 512
_TN = 4096
_TM = 4096


def _ceil_to(x, m):
    return ((x + m - 1) // m) * m


def _min_fold(vs):
    if len(vs) == 1:
        return vs[0]
    h = len(vs) // 2
    return jnp.minimum(_min_fold(vs[:h]), _min_fold(vs[h:]))


def _hilo(x):
    xi = lax.bitcast_convert_type(x, jnp.uint32)
    hi = lax.bitcast_convert_type(
        (xi + jnp.uint32(0x8000)) & jnp.uint32(0xFFFF0000), jnp.float32)
    lo = x - hi
    return hi.astype(jnp.bfloat16), lo.astype(jnp.bfloat16)


def _pack_features(xyz, val, n_pad, side):
    P, _, N = xyz.shape
    pad = n_pad - N
    if pad:
        xyz = jnp.pad(xyz, ((0, 0), (0, 0), (0, pad)))
        val = jnp.pad(val, ((0, 0), (0, pad)))
    w = xyz if side == "p" else -2.0 * xyz
    sq = jnp.sum(xyz * xyz, axis=1, keepdims=True)
    w_hi, w_lo = _hilo(w)
    s_hi, s_lo = _hilo(sq)
    pen = (_BIG * (1.0 - val))[:, None, :].astype(jnp.bfloat16)
    one = jnp.ones_like(pen)
    zero = jnp.zeros_like(pen)
    if side == "p":
        feats = [w_hi, w_hi, w_lo, s_hi, s_lo, one, one, pen, one, zero]
    else:
        feats = [w_hi, w_lo, w_hi, one, one, s_hi, s_lo, one, pen, zero]
    return jnp.concatenate(feats, axis=1)


def _tile_kernel(cnt_ref, a_ref, b_ref, d1_ref, d2_ref, rmin_ref, qmin_ref):
    p = pl.program_id(0)
    i = pl.program_id(1)
    j = pl.program_id(2)
    last_i = i == pl.num_programs(1) - 1
    last_j = j == pl.num_programs(2) - 1
    tn = a_ref.shape[2]
    tm = b_ref.shape[2]
    nr = tn // _RS
    nc = tm // _CS

    @pl.when((i == 0) & (j == 0))
    def _():
        d1_ref[...] = jnp.zeros_like(d1_ref)

    @pl.when(j == 0)
    def _():
        rmin_ref[...] = jnp.full(rmin_ref.shape, _INF, jnp.bfloat16)

    @pl.when(i == 0)
    def _():
        qmin_ref[:, pl.ds(pl.multiple_of(j * tm, _CS), tm)] = jnp.full(
            (16, tm), _INF, jnp.bfloat16)

    @pl.when((i * tn < cnt_ref[p, 0]) & (j * tm < cnt_ref[p, 1]))
    def _():
        qacc = [qmin_ref[:, pl.ds(pl.multiple_of(j * tm + c * _CS, _CS), _CS)]
                for c in range(nc)]

        for r in range(nr):
            a = a_ref[0, :, r * _RS:(r + 1) * _RS]
            rmin = rmin_ref[r * _RS:(r + 1) * _RS, :]
            for c in range(nc):
                b = b_ref[0, :, c * _CS:(c + 1) * _CS]
                d = lax.dot_general(
                    a, b,
                    dimension_numbers=(((0,), (0,)), ((), ())),
                    preferred_element_type=jnp.float32)
                d = d.astype(jnp.bfloat16)
                rmin = jnp.minimum(rmin, _min_fold(
                    [d[:, k * 128:(k + 1) * 128] for k in range(_CS // 128)]))
                qacc[c] = jnp.minimum(qacc[c], _min_fold(
                    [d[k * 16:(k + 1) * 16, :] for k in range(_RS // 16)]))
            rmin_ref[r * _RS:(r + 1) * _RS, :] = rmin

        for c in range(nc):
            qmin_ref[:, pl.ds(pl.multiple_of(j * tm + c * _CS, _CS), _CS)] = (
                qacc[c])

    @pl.when(last_j)
    def _():
        rm = jnp.min(rmin_ref[...].astype(jnp.float32),
                     axis=1, keepdims=True)
        d1_ref[...] = d1_ref[...] + jnp.reshape(
            jnp.sum(jnp.where(rm < _THRESH, rm, 0.0)), (1, 1, 1))

    @pl.when(last_i & last_j)
    def _():
        qcol = jnp.min(qmin_ref[...].astype(jnp.float32),
                       axis=0, keepdims=True)
        d2_ref[...] = jnp.reshape(
            jnp.sum(jnp.where(qcol < _THRESH, qcol, 0.0)), (1, 1, 1))


def _chamfer_sums_local(p_xyz, p_val, q_xyz, q_val):
    num_pairs, _, n_p = p_xyz.shape
    n_q = q_xyz.shape[2]

    tn = min(_TN, _ceil_to(n_p, _RS))
    tm = min(_TM, _ceil_to(n_q, _CS))
    np_pad = _ceil_to(n_p, tn)
    nq_pad = _ceil_to(n_q, tm)

    lhs = _pack_features(p_xyz, p_val, np_pad, "p")
    rhs = _pack_features(q_xyz, q_val, nq_pad, "q")
    counts = jnp.stack(
        [jnp.sum(p_val, axis=1), jnp.sum(q_val, axis=1)],
        axis=1).astype(jnp.int32)

    grid = (num_pairs, np_pad // tn, nq_pad // tm)
    d1, d2 = pl.pallas_call(
        _tile_kernel,
        out_shape=(jax.ShapeDtypeStruct((num_pairs, 1, 1), jnp.float32),
                   jax.ShapeDtypeStruct((num_pairs, 1, 1), jnp.float32)),
        grid_spec=pltpu.PrefetchScalarGridSpec(
            num_scalar_prefetch=1,
            grid=grid,
            in_specs=[
                pl.BlockSpec((1, 16, tn), lambda p, i, j, c: (p, 0, i)),
                pl.BlockSpec((1, 16, tm), lambda p, i, j, c: (p, 0, j)),
            ],
            out_specs=[
                pl.BlockSpec((1, 1, 1), lambda p, i, j, c: (p, 0, 0)),
                pl.BlockSpec((1, 1, 1), lambda p, i, j, c: (p, 0, 0)),
            ],
            scratch_shapes=[
                pltpu.VMEM((tn, 128), jnp.bfloat16),
                pltpu.VMEM((16, nq_pad), jnp.bfloat16),
            ]),
        compiler_params=pltpu.CompilerParams(
            dimension_semantics=("parallel", "arbitrary", "arbitrary"),
            vmem_limit_bytes=64 * 1024 * 1024),
    )(counts, lhs, rhs)
    return d1[:, 0, 0], d2[:, 0, 0]


def _chamfer_sums(p_xyz, p_val, q_xyz, q_val):
    devs = jax.devices()
    num_pairs = p_xyz.shape[0]
    n_dev = 2 if (len(devs) >= 2 and num_pairs % 2 == 0) else 1
    if n_dev == 1:
        return _chamfer_sums_local(p_xyz, p_val, q_xyz, q_val)
    try:
        shard_map = jax.shard_map
    except AttributeError:
        from jax.experimental.shard_map import shard_map
    mesh = jax.sharding.Mesh(_onp.array(devs[:n_dev]), ("d",))
    spec = jax.sharding.PartitionSpec("d")
    fn = shard_map(_chamfer_sums_local, mesh=mesh,
                   in_specs=(spec, spec, spec, spec),
                   out_specs=(spec, spec), check_vma=False)
    return fn(p_xyz, p_val, q_xyz, q_val)


def _ray_dirs(h, w, fov_up_deg=3.0, fov_down_deg=-25.0):
    fov_up = math.radians(fov_up_deg)
    fov_down = math.radians(fov_down_deg)
    az = -math.pi + (jnp.arange(w, dtype=jnp.float32) + 0.5) * (
        2.0 * math.pi / w)
    el = fov_up - (jnp.arange(h, dtype=jnp.float32) + 0.5) * (
        (fov_up - fov_down) / h)
    az = az[None, :]
    el = el[:, None]
    x = jnp.cos(el) * jnp.cos(az)
    y = jnp.cos(el) * jnp.sin(az)
    z = jnp.sin(el) * jnp.ones_like(az)
    return jnp.stack([x, y, z], axis=0)


def kernel(rv, mask_logits, target):
    rv = rv.astype(jnp.float32)
    logits = mask_logits.astype(jnp.float32)
    target = target.astype(jnp.float32)
    b, t, h, w = rv.shape
    dirs = _ray_dirs(h, w)

    masked_rv = jnp.where(logits > 0.0, rv, -1.0)

    p_xyz = masked_rv[:, :, None] * dirs[None, None]
    p_val = (masked_rv > 0.0).astype(jnp.float32)
    q_xyz = target[:, 1:4]
    q_val = (target[:, 0] > 0.0).astype(jnp.float32)

    p_xyz = p_xyz.transpose(1, 0, 2, 3, 4).reshape(t * b, 3, h * w)
    p_val = p_val.transpose(1, 0, 2, 3).reshape(t * b, h * w)
    q_xyz = q_xyz.transpose(2, 0, 1, 3, 4).reshape(t * b, 3, h * w)
    q_val = q_val.transpose(1, 0, 2, 3).reshape(t * b, h * w)

    n_p = jnp.sum(p_val, axis=1)
    n_q = jnp.sum(q_val, axis=1)

    def _compact(xyz, val):
        key, x, y, z = lax.sort(
            (1.0 - val, xyz[:, 0], xyz[:, 1], xyz[:, 2]),
            dimension=1, is_stable=False, num_keys=1)
        return jnp.stack([x, y, z], axis=1), 1.0 - key

    p_xyz, p_val = _compact(p_xyz, p_val)
    q_xyz, q_val = _compact(q_xyz, q_val)
    d1_sum, d2_sum = _chamfer_sums(p_xyz, p_val, q_xyz, q_val)
    combined = d1_sum / jnp.maximum(n_p, 1.0) + d2_sum / jnp.maximum(n_q, 1.0)
    chamfer_distances_tensor = combined.reshape(t, b)
    chamfer_distances = {s: jnp.mean(chamfer_distances_tensor[s])
                         for s in range(t)}
    return chamfer_distances, chamfer_distances_tensor
